# baseline (speedup 1.0000x reference)
import functools

import jax
import jax.numpy as jnp
from jax import lax
from jax.experimental import pallas as pl
from jax.experimental.pallas import tpu as pltpu
from jax.experimental.pallas import tpu_sc as plsc

NC = 2
NS = 16
NW = NC * NS
LANES = 16
B = 128
BIAS_NUM = 63
BIAS_DEN = 80


def _fill_vec(ref, val, n):
    v = jnp.full((LANES,), val, dtype=ref.dtype)

    def body(i, c):
        ref[pl.ds(i * LANES, LANES)] = v
        return c

    lax.fori_loop(0, n // LANES, body, 0)


def _deg_kernel(n_pad, cpw, zs, dst_hbm, degp_hbm, idx_v, ones_v, z_v, deg_sh,
                sem):
    c = lax.axis_index("c")
    s = lax.axis_index("s")
    wid = s * NC + c

    _fill_vec(ones_v, 1.0, B)
    _fill_vec(z_v, 0.0, zs)
    pltpu.sync_copy(z_v, deg_sh.at[pl.ds(s * zs, zs)])
    plsc.subcore_barrier()

    gbase = wid * cpw

    def chunk(ci, carry):
        pltpu.sync_copy(dst_hbm.at[gbase + ci, 1], idx_v)
        pltpu.sync_copy(ones_v, deg_sh.at[idx_v], add=True)
        return carry

    lax.fori_loop(0, cpw, chunk, 0)
    plsc.subcore_barrier()
    pltpu.sync_copy(deg_sh.at[pl.ds(s * zs, zs)],
                    degp_hbm.at[pl.ds(c * n_pad + s * zs, zs)])


def _edge_kernel(n_pad, n0, n1, zs, d, ei_hbm, xs_hbm, outp_hbm,
                 ei_a, ei_b, rows_a, rows_b, z_v, out_sh, ga, gb, sa, sb):
    c = lax.axis_index("c")
    s = lax.axis_index("s")

    with jax.named_scope("acc_zero"):
        def zrow(i, carry):
            def zcol(j, cc):
                z_v[i, pl.ds(j * LANES, LANES)] = jnp.zeros((LANES,),
                                                            jnp.float32)
                return cc

            lax.fori_loop(0, d // LANES, zcol, 0)
            return carry

        lax.fori_loop(0, 64, zrow, 0)

        def zcopy(t, carry):
            pltpu.sync_copy(z_v, out_sh.at[pl.ds(s * zs + t * 64, 64)])
            return carry

        lax.fori_loop(0, zs // 64, zcopy, 0)
        plsc.subcore_barrier()

    gbase = s * n0

    def g_start(ei, rows, sem):
        pltpu.async_copy(xs_hbm.at[ei.at[0]], rows, sem)

    def g_wait(ei, rows, sem):
        pltpu.make_async_copy(xs_hbm.at[ei.at[0]], rows, sem).wait()

    def s_start(ei, rows, sem):
        pltpu.async_copy(rows, out_sh.at[ei.at[1]], sem, add=True)

    def s_wait(ei, rows, sem):
        pltpu.make_async_copy(rows, out_sh.at[ei.at[1]], sem).wait()

    @pl.when(c == 0)
    def _edge_loop():
        with jax.named_scope("chunk_loop"):
            pltpu.sync_copy(ei_hbm.at[gbase], ei_a)
            g_start(ei_a, rows_a, ga)
            pltpu.sync_copy(ei_hbm.at[gbase + 1], ei_b)
            g_start(ei_b, rows_b, gb)
            g_wait(ei_a, rows_a, ga)
            s_start(ei_a, rows_a, sa)

            def body(t, carry):
                g0 = gbase + 2 * t
                s_wait(ei_a, rows_a, sa)
                pltpu.sync_copy(ei_hbm.at[g0], ei_a)
                g_start(ei_a, rows_a, ga)
                g_wait(ei_b, rows_b, gb)
                s_start(ei_b, rows_b, sb)
                s_wait(ei_b, rows_b, sb)
                pltpu.sync_copy(ei_hbm.at[g0 + 1], ei_b)
                g_start(ei_b, rows_b, gb)
                g_wait(ei_a, rows_a, ga)
                s_start(ei_a, rows_a, sa)
                return carry

            lax.fori_loop(1, n0 // 2, body, 0)

            g_wait(ei_b, rows_b, gb)
            s_start(ei_b, rows_b, sb)
            s_wait(ei_a, rows_a, sa)
            s_wait(ei_b, rows_b, sb)

        with jax.named_scope("copyout"):
            plsc.subcore_barrier()
            pltpu.sync_copy(out_sh.at[pl.ds(s * zs, zs)],
                            outp_hbm.at[pl.ds(s * zs, zs)])


def _dis_from_parts(dp_ref):
    deg = dp_ref[0, :] + dp_ref[1, :]
    return jnp.where(deg > 0, lax.rsqrt(jnp.maximum(deg, 1.0)), 0.0)


def _scale_kernel(dp_ref, x_ref, xs_ref):
    dis = _dis_from_parts(dp_ref)
    xs_ref[...] = x_ref[...] * dis[:, None]


def _combine_kernel(p_ref, dp_ref, o_ref):
    dis = _dis_from_parts(dp_ref)
    o_ref[...] = p_ref[...] * dis[:, None]


@jax.jit
def kernel(x, edge_index):
    n, d = x.shape
    e = edge_index.shape[1]

    n_pad = ((n + NS * LANES - 1) // (NS * LANES)) * (NS * LANES)
    zs = n_pad // NS
    cpw = -(-e // (NW * B))
    cpw += cpw % 2
    e_pad = cpw * B * NW
    sac = n_pad - 1

    n0 = cpw * NC
    n1 = 0

    ei = (
        jnp.full((2, e_pad), sac, jnp.int32)
        .at[:, :e].set(edge_index)
        .reshape(2, NW * cpw, B)
        .transpose(1, 0, 2)
    )
    x_pad = jnp.zeros((n_pad, d), x.dtype).at[:n].set(x)

    mesh = plsc.VectorSubcoreMesh(core_axis_name="c", subcore_axis_name="s",
                                  num_cores=NC, num_subcores=NS)

    deg_parts = pl.kernel(
        functools.partial(_deg_kernel, n_pad, cpw, zs),
        out_type=jax.ShapeDtypeStruct((NC * n_pad,), jnp.float32),
        mesh=mesh,
        scratch_types=[
            pltpu.VMEM((B,), jnp.int32),
            pltpu.VMEM((B,), jnp.float32),
            pltpu.VMEM((zs,), jnp.float32),
            pltpu.VMEM_SHARED((n_pad,), jnp.float32),
            pltpu.SemaphoreType.DMA,
        ],
    )(ei)
    deg_parts = deg_parts.reshape(NC, n_pad)

    rb = 1024
    grid = n_pad // rb
    xs = pl.pallas_call(
        _scale_kernel,
        grid=(grid,),
        in_specs=[
            pl.BlockSpec((NC, rb), lambda i: (0, i)),
            pl.BlockSpec((rb, d), lambda i: (i, 0)),
        ],
        out_specs=pl.BlockSpec((rb, d), lambda i: (i, 0)),
        out_shape=jax.ShapeDtypeStruct((n_pad, d), jnp.float32),
    )(deg_parts, x_pad)

    out_parts = pl.kernel(
        functools.partial(_edge_kernel, n_pad, n0, n1, zs, d),
        out_type=jax.ShapeDtypeStruct((n_pad, d), jnp.float32),
        mesh=mesh,
        scratch_types=[
            pltpu.VMEM((2, B), jnp.int32),
            pltpu.VMEM((2, B), jnp.int32),
            pltpu.VMEM((B, d), jnp.float32),
            pltpu.VMEM((B, d), jnp.float32),
            pltpu.VMEM((64, d), jnp.float32),
            pltpu.VMEM_SHARED((n_pad, d), jnp.float32),
            pltpu.SemaphoreType.DMA,
            pltpu.SemaphoreType.DMA,
            pltpu.SemaphoreType.DMA,
            pltpu.SemaphoreType.DMA,
        ],
    )(ei, xs)

    out_pad = pl.pallas_call(
        _combine_kernel,
        grid=(grid,),
        in_specs=[
            pl.BlockSpec((rb, d), lambda i: (i, 0)),
            pl.BlockSpec((NC, rb), lambda i: (0, i)),
        ],
        out_specs=pl.BlockSpec((rb, d), lambda i: (i, 0)),
        out_shape=jax.ShapeDtypeStruct((n_pad, d), jnp.float32),
    )(out_parts, deg_parts)

    return out_pad[:n]


# speedup vs baseline: 3.1388x; 3.1388x over previous
import functools

import jax
import jax.numpy as jnp
from jax import lax
from jax.experimental import pallas as pl
from jax.experimental.pallas import tpu as pltpu
from jax.experimental.pallas import tpu_sc as plsc

NC = 2
NS = 16
NW = NC * NS
LANES = 16
B = 128
BIAS_NUM = 63
BIAS_DEN = 80


def _fill_vec(ref, val, n):
    v = jnp.full((LANES,), val, dtype=ref.dtype)

    def body(i, c):
        ref[pl.ds(i * LANES, LANES)] = v
        return c

    lax.fori_loop(0, n // LANES, body, 0)


def _deg_kernel(n_pad, cpw, zs, dst_hbm, degp_hbm, idx_v, ones_v, z_v, deg_sh,
                sem):
    c = lax.axis_index("c")
    s = lax.axis_index("s")
    wid = s * NC + c

    _fill_vec(ones_v, 1.0, B)
    _fill_vec(z_v, 0.0, zs)
    pltpu.sync_copy(z_v, deg_sh.at[pl.ds(s * zs, zs)])
    plsc.subcore_barrier()

    gbase = wid * cpw

    def chunk(ci, carry):
        pltpu.sync_copy(dst_hbm.at[gbase + ci, 1], idx_v)
        pltpu.sync_copy(ones_v, deg_sh.at[idx_v], add=True)
        return carry

    lax.fori_loop(0, cpw, chunk, 0)
    plsc.subcore_barrier()
    pltpu.sync_copy(deg_sh.at[pl.ds(s * zs, zs)],
                    degp_hbm.at[pl.ds(c * n_pad + s * zs, zs)])


def _edge_kernel(n_pad, n0, n1, zs, d, ei_hbm, xs_hbm, outp_hbm,
                 ei_a, ei_b, rows_a, rows_b, z_v, out_sh, ga, gb, sa, sb):
    c = lax.axis_index("c")
    s = lax.axis_index("s")

    with jax.named_scope("acc_zero"):
        def zrow(i, carry):
            def zcol(j, cc):
                z_v[i, pl.ds(j * LANES, LANES)] = jnp.zeros((LANES,),
                                                            jnp.float32)
                return cc

            lax.fori_loop(0, d // LANES, zcol, 0)
            return carry

        lax.fori_loop(0, 64, zrow, 0)

        def zcopy(t, carry):
            pltpu.sync_copy(z_v, out_sh.at[pl.ds(s * zs + t * 64, 64)])
            return carry

        lax.fori_loop(0, zs // 64, zcopy, 0)
        plsc.subcore_barrier()

    wid = s * NC + c
    gbase = wid * n0

    def g_start(ei, rows, sem):
        pltpu.async_copy(xs_hbm.at[ei.at[0]], rows, sem)

    def g_wait(ei, rows, sem):
        pltpu.make_async_copy(xs_hbm.at[ei.at[0]], rows, sem).wait()

    def s_start(ei, rows, sem):
        pltpu.async_copy(rows, out_sh.at[ei.at[1]], sem, add=True)

    def s_wait(ei, rows, sem):
        pltpu.make_async_copy(rows, out_sh.at[ei.at[1]], sem).wait()

    if True:
        with jax.named_scope("chunk_loop"):
            pltpu.sync_copy(ei_hbm.at[gbase], ei_a)
            g_start(ei_a, rows_a, ga)
            pltpu.sync_copy(ei_hbm.at[gbase + 1], ei_b)
            g_start(ei_b, rows_b, gb)
            g_wait(ei_a, rows_a, ga)
            s_start(ei_a, rows_a, sa)

            def body(t, carry):
                g0 = gbase + 2 * t
                s_wait(ei_a, rows_a, sa)
                pltpu.sync_copy(ei_hbm.at[g0], ei_a)
                g_start(ei_a, rows_a, ga)
                g_wait(ei_b, rows_b, gb)
                s_start(ei_b, rows_b, sb)
                s_wait(ei_b, rows_b, sb)
                pltpu.sync_copy(ei_hbm.at[g0 + 1], ei_b)
                g_start(ei_b, rows_b, gb)
                g_wait(ei_a, rows_a, ga)
                s_start(ei_a, rows_a, sa)
                return carry

            lax.fori_loop(1, n0 // 2, body, 0)

            g_wait(ei_b, rows_b, gb)
            s_start(ei_b, rows_b, sb)
            s_wait(ei_a, rows_a, sa)
            s_wait(ei_b, rows_b, sb)

        with jax.named_scope("copyout"):
            plsc.subcore_barrier()
            pltpu.sync_copy(out_sh.at[pl.ds(s * zs, zs)],
                            outp_hbm.at[pl.ds(c * n_pad + s * zs, zs)])


def _dis_from_parts(dp_ref):
    deg = dp_ref[0, :] + dp_ref[1, :]
    return jnp.where(deg > 0, lax.rsqrt(jnp.maximum(deg, 1.0)), 0.0)


def _scale_kernel(dp_ref, x_ref, xs_ref):
    dis = _dis_from_parts(dp_ref)
    xs_ref[...] = x_ref[...] * dis[:, None]


def _combine_kernel(p_ref, dp_ref, o_ref):
    dis = _dis_from_parts(dp_ref)
    o_ref[...] = (p_ref[0] + p_ref[1]) * dis[:, None]


@jax.jit
def kernel(x, edge_index):
    n, d = x.shape
    e = edge_index.shape[1]

    n_pad = ((n + NS * LANES - 1) // (NS * LANES)) * (NS * LANES)
    zs = n_pad // NS
    cpw = -(-e // (NW * B))
    cpw += cpw % 2
    e_pad = cpw * B * NW
    sac = n_pad - 1

    n0 = cpw
    n1 = 0

    pad_idx = n + (jnp.arange(e_pad - e, dtype=jnp.int32) % (n_pad - n))
    ei = (
        jnp.concatenate(
            [edge_index.astype(jnp.int32),
             jnp.broadcast_to(pad_idx, (2, e_pad - e))], axis=1)
        .reshape(2, NW * cpw, B)
        .transpose(1, 0, 2)
    )
    x_pad = jnp.zeros((n_pad, d), x.dtype).at[:n].set(x)

    mesh = plsc.VectorSubcoreMesh(core_axis_name="c", subcore_axis_name="s",
                                  num_cores=NC, num_subcores=NS)

    deg_parts = pl.kernel(
        functools.partial(_deg_kernel, n_pad, cpw, zs),
        out_type=jax.ShapeDtypeStruct((NC * n_pad,), jnp.float32),
        mesh=mesh,
        scratch_types=[
            pltpu.VMEM((B,), jnp.int32),
            pltpu.VMEM((B,), jnp.float32),
            pltpu.VMEM((zs,), jnp.float32),
            pltpu.VMEM_SHARED((n_pad,), jnp.float32),
            pltpu.SemaphoreType.DMA,
        ],
    )(ei)
    deg_parts = deg_parts.reshape(NC, n_pad)

    rb = 1024
    grid = n_pad // rb
    xs = pl.pallas_call(
        _scale_kernel,
        grid=(grid,),
        in_specs=[
            pl.BlockSpec((NC, rb), lambda i: (0, i)),
            pl.BlockSpec((rb, d), lambda i: (i, 0)),
        ],
        out_specs=pl.BlockSpec((rb, d), lambda i: (i, 0)),
        out_shape=jax.ShapeDtypeStruct((n_pad, d), jnp.float32),
    )(deg_parts, x_pad)

    out_parts = pl.kernel(
        functools.partial(_edge_kernel, n_pad, n0, n1, zs, d),
        out_type=jax.ShapeDtypeStruct((NC * n_pad, d), jnp.float32),
        mesh=mesh,
        scratch_types=[
            pltpu.VMEM((2, B), jnp.int32),
            pltpu.VMEM((2, B), jnp.int32),
            pltpu.VMEM((B, d), jnp.float32),
            pltpu.VMEM((B, d), jnp.float32),
            pltpu.VMEM((64, d), jnp.float32),
            pltpu.VMEM_SHARED((n_pad, d), jnp.float32),
            pltpu.SemaphoreType.DMA,
            pltpu.SemaphoreType.DMA,
            pltpu.SemaphoreType.DMA,
            pltpu.SemaphoreType.DMA,
        ],
    )(ei, xs)
    out_parts = out_parts.reshape(NC, n_pad, d)

    out_pad = pl.pallas_call(
        _combine_kernel,
        grid=(grid,),
        in_specs=[
            pl.BlockSpec((NC, rb, d), lambda i: (0, i, 0)),
            pl.BlockSpec((NC, rb), lambda i: (0, i)),
        ],
        out_specs=pl.BlockSpec((rb, d), lambda i: (i, 0)),
        out_shape=jax.ShapeDtypeStruct((n_pad, d), jnp.float32),
    )(out_parts, deg_parts)

    return out_pad[:n]


# speedup vs baseline: 3.2515x; 1.0359x over previous
import functools

import jax
import jax.numpy as jnp
from jax import lax
from jax.experimental import pallas as pl
from jax.experimental.pallas import tpu as pltpu
from jax.experimental.pallas import tpu_sc as plsc

NC = 2
NS = 16
NW = NC * NS
LANES = 16
B = 128
H = 1


def _fill_vec(ref, val, n):
    v = jnp.full((LANES,), val, dtype=ref.dtype)

    def body(i, c):
        ref[pl.ds(i * LANES, LANES)] = v
        return c

    lax.fori_loop(0, n // LANES, body, 0)


def _deg_kernel(n_pad, cpw, zs, ei_hbm, degp_hbm, idx_a, idx_b, ones_v, z_v,
                deg_sh, sa, sb):
    c = lax.axis_index("c")
    s = lax.axis_index("s")
    wid = s * NC + c

    _fill_vec(ones_v, 1.0, B)
    _fill_vec(z_v, 0.0, zs)
    pltpu.sync_copy(z_v, deg_sh.at[pl.ds(s * zs, zs)])
    plsc.subcore_barrier()

    gbase = wid * cpw

    def s_start(idx, sem):
        pltpu.async_copy(ones_v, deg_sh.at[idx], sem, add=True)

    def s_wait(idx, sem):
        pltpu.make_async_copy(ones_v, deg_sh.at[idx], sem).wait()

    pltpu.sync_copy(ei_hbm.at[gbase, 1], idx_a)
    s_start(idx_a, sa)
    pltpu.sync_copy(ei_hbm.at[gbase + 1, 1], idx_b)
    s_start(idx_b, sb)

    def chunk(t, carry):
        g0 = gbase + 2 * t
        s_wait(idx_a, sa)
        pltpu.sync_copy(ei_hbm.at[g0, 1], idx_a)
        s_start(idx_a, sa)
        s_wait(idx_b, sb)
        pltpu.sync_copy(ei_hbm.at[g0 + 1, 1], idx_b)
        s_start(idx_b, sb)
        return carry

    lax.fori_loop(1, cpw // 2, chunk, 0)
    s_wait(idx_a, sa)
    s_wait(idx_b, sb)

    plsc.subcore_barrier()
    pltpu.sync_copy(deg_sh.at[pl.ds(s * zs, zs)],
                    degp_hbm.at[pl.ds(c * n_pad + s * zs, zs)])


def _edge_kernel(n_pad, cpw, zs, d, ei_hbm, xs_hbm, outp_hbm,
                 ei_a, ei_b, rows_a, rows_b, z_v, out_sh, ga, gb, sa, sb):
    c = lax.axis_index("c")
    s = lax.axis_index("s")

    with jax.named_scope("acc_zero"):
        def zrow(i, carry):
            def zcol(j, cc):
                z_v[i, pl.ds(j * LANES, LANES)] = jnp.zeros((LANES,),
                                                            jnp.float32)
                return cc

            lax.fori_loop(0, d // LANES, zcol, 0)
            return carry

        lax.fori_loop(0, 64, zrow, 0)

        def zcopy(t, carry):
            pltpu.sync_copy(z_v, out_sh.at[pl.ds(s * zs + t * 64, 64)])
            return carry

        lax.fori_loop(0, zs // 64, zcopy, 0)
        plsc.subcore_barrier()

    wid = s * NC + c
    gbase = wid * cpw

    def g_start(ei, rows, sem):
        for h in range(H):
            pltpu.async_copy(xs_hbm.at[ei.at[h, 0]],
                             rows.at[pl.ds(h * B, B)], sem)

    def g_wait(ei, rows, sem):
        for h in range(H):
            pltpu.make_async_copy(xs_hbm.at[ei.at[h, 0]],
                                  rows.at[pl.ds(h * B, B)], sem).wait()

    def s_start(ei, rows, sem):
        for h in range(H):
            pltpu.async_copy(rows.at[pl.ds(h * B, B)],
                             out_sh.at[ei.at[h, 1]], sem, add=True)

    def s_wait(ei, rows, sem):
        for h in range(H):
            pltpu.make_async_copy(rows.at[pl.ds(h * B, B)],
                                  out_sh.at[ei.at[h, 1]], sem).wait()

    with jax.named_scope("chunk_loop"):
        pltpu.sync_copy(ei_hbm.at[pl.ds(gbase, H)], ei_a)
        g_start(ei_a, rows_a, ga)
        pltpu.sync_copy(ei_hbm.at[pl.ds(gbase + H, H)], ei_b)
        g_start(ei_b, rows_b, gb)
        g_wait(ei_a, rows_a, ga)
        s_start(ei_a, rows_a, sa)

        def body(t, carry):
            g0 = gbase + 2 * H * t
            s_wait(ei_a, rows_a, sa)
            pltpu.sync_copy(ei_hbm.at[pl.ds(g0, H)], ei_a)
            g_start(ei_a, rows_a, ga)
            g_wait(ei_b, rows_b, gb)
            s_start(ei_b, rows_b, sb)
            s_wait(ei_b, rows_b, sb)
            pltpu.sync_copy(ei_hbm.at[pl.ds(g0 + H, H)], ei_b)
            g_start(ei_b, rows_b, gb)
            g_wait(ei_a, rows_a, ga)
            s_start(ei_a, rows_a, sa)
            return carry

        lax.fori_loop(1, cpw // (2 * H), body, 0)

        g_wait(ei_b, rows_b, gb)
        s_start(ei_b, rows_b, sb)
        s_wait(ei_a, rows_a, sa)
        s_wait(ei_b, rows_b, sb)

    with jax.named_scope("copyout"):
        plsc.subcore_barrier()
        pltpu.sync_copy(out_sh.at[pl.ds(s * zs, zs)],
                        outp_hbm.at[pl.ds(c * n_pad + s * zs, zs)])


def _dis_from_parts(dp_ref):
    deg = dp_ref[0, :] + dp_ref[1, :]
    return jnp.where(deg > 0, lax.rsqrt(jnp.maximum(deg, 1.0)), 0.0)


def _scale_kernel(dp_ref, x_ref, xs_ref):
    dis = _dis_from_parts(dp_ref)
    xs_ref[...] = x_ref[...] * dis[:, None]


def _combine_kernel(p_ref, dp_ref, o_ref):
    dis = _dis_from_parts(dp_ref)
    o_ref[...] = (p_ref[0] + p_ref[1]) * dis[:, None]


@jax.jit
def kernel(x, edge_index):
    n, d = x.shape
    e = edge_index.shape[1]

    n_pad = (n // (NS * LANES) + 1) * (NS * LANES)
    zs = n_pad // NS
    cpw = -(-e // (NW * B))
    cpw += (-cpw) % (2 * H)
    e_pad = cpw * B * NW

    pad_idx = n + (jnp.arange(e_pad - e, dtype=jnp.int32) % (n_pad - n))
    ei = (
        jnp.concatenate(
            [edge_index.astype(jnp.int32),
             jnp.broadcast_to(pad_idx, (2, e_pad - e))], axis=1)
        .reshape(2, NW * cpw, B)
        .transpose(1, 0, 2)
    )
    x_pad = jnp.zeros((n_pad, d), x.dtype).at[:n].set(x)

    mesh = plsc.VectorSubcoreMesh(core_axis_name="c", subcore_axis_name="s",
                                  num_cores=NC, num_subcores=NS)

    deg_parts = pl.kernel(
        functools.partial(_deg_kernel, n_pad, cpw, zs),
        out_type=jax.ShapeDtypeStruct((NC * n_pad,), jnp.float32),
        mesh=mesh,
        scratch_types=[
            pltpu.VMEM((B,), jnp.int32),
            pltpu.VMEM((B,), jnp.int32),
            pltpu.VMEM((B,), jnp.float32),
            pltpu.VMEM((zs,), jnp.float32),
            pltpu.VMEM_SHARED((n_pad,), jnp.float32),
            pltpu.SemaphoreType.DMA,
            pltpu.SemaphoreType.DMA,
        ],
    )(ei)
    deg_parts = deg_parts.reshape(NC, n_pad)

    rb = 1024
    xs = pl.pallas_call(
        _scale_kernel,
        grid=(n_pad // rb,),
        in_specs=[
            pl.BlockSpec((NC, rb), lambda i: (0, i)),
            pl.BlockSpec((rb, d), lambda i: (i, 0)),
        ],
        out_specs=pl.BlockSpec((rb, d), lambda i: (i, 0)),
        out_shape=jax.ShapeDtypeStruct((n_pad, d), jnp.float32),
    )(deg_parts, x_pad)

    out_parts = pl.kernel(
        functools.partial(_edge_kernel, n_pad, cpw, zs, d),
        out_type=jax.ShapeDtypeStruct((NC * n_pad, d), jnp.float32),
        mesh=mesh,
        scratch_types=[
            pltpu.VMEM((H, 2, B), jnp.int32),
            pltpu.VMEM((H, 2, B), jnp.int32),
            pltpu.VMEM((H * B, d), jnp.float32),
            pltpu.VMEM((H * B, d), jnp.float32),
            pltpu.VMEM((64, d), jnp.float32),
            pltpu.VMEM_SHARED((n_pad, d), jnp.float32),
            pltpu.SemaphoreType.DMA,
            pltpu.SemaphoreType.DMA,
            pltpu.SemaphoreType.DMA,
            pltpu.SemaphoreType.DMA,
        ],
    )(ei, xs)
    out_parts = out_parts.reshape(NC, n_pad, d)

    out_pad = pl.pallas_call(
        _combine_kernel,
        grid=(n_pad // rb,),
        in_specs=[
            pl.BlockSpec((NC, rb, d), lambda i: (0, i, 0)),
            pl.BlockSpec((NC, rb), lambda i: (0, i)),
        ],
        out_specs=pl.BlockSpec((rb, d), lambda i: (i, 0)),
        out_shape=jax.ShapeDtypeStruct((n_pad, d), jnp.float32),
    )(out_parts, deg_parts)

    return out_pad[:n]


# speedup vs baseline: 3.7289x; 1.1468x over previous
import functools

import jax
import jax.numpy as jnp
from jax import lax
from jax.experimental import pallas as pl
from jax.experimental.pallas import tpu as pltpu
from jax.experimental.pallas import tpu_sc as plsc

NC = 2
NS = 16
NW = NC * NS
LANES = 16
B = 128
H = 1
K = 8


def _fill_vec(ref, val, n):
    v = jnp.full((LANES,), val, dtype=ref.dtype)

    def body(i, c):
        ref[pl.ds(i * LANES, LANES)] = v
        return c

    lax.fori_loop(0, n // LANES, body, 0)


def _deg_kernel(n_pad, cpw, zs, dstc_hbm, degp_hbm, idx_a, idx_b, ones_v, z_v,
                deg_sh, la, lb, sa, sb):
    c = lax.axis_index("c")
    s = lax.axis_index("s")
    wid = s * NC + c

    _fill_vec(ones_v, 1.0, B)
    _fill_vec(z_v, 0.0, zs)
    pltpu.sync_copy(z_v, deg_sh.at[pl.ds(s * zs, zs)])
    plsc.subcore_barrier()

    gbase = wid * cpw

    def l_start(buf, g0, sem):
        pltpu.async_copy(dstc_hbm.at[pl.ds(g0, K)], buf, sem)

    def l_wait(buf, sem):
        pltpu.make_async_copy(dstc_hbm.at[pl.ds(gbase, K)], buf, sem).wait()

    def s_all(buf, sem):
        for j in range(K):
            pltpu.async_copy(ones_v, deg_sh.at[buf.at[j]], sem, add=True)

    def s_drain(buf, sem):
        for j in range(K):
            pltpu.make_async_copy(ones_v, deg_sh.at[buf.at[j]], sem).wait()

    nslab = cpw // K
    l_start(idx_a, gbase, la)
    l_start(idx_b, gbase + K, lb)
    l_wait(idx_a, la)
    s_all(idx_a, sa)

    def slab(t, carry):
        g0 = gbase + 2 * K * t
        s_drain(idx_a, sa)
        l_start(idx_a, g0, la)
        l_wait(idx_b, lb)
        s_all(idx_b, sb)
        s_drain(idx_b, sb)
        l_start(idx_b, g0 + K, lb)
        l_wait(idx_a, la)
        s_all(idx_a, sa)
        return carry

    lax.fori_loop(1, nslab // 2, slab, 0)
    l_wait(idx_b, lb)
    s_all(idx_b, sb)
    s_drain(idx_a, sa)
    s_drain(idx_b, sb)

    plsc.subcore_barrier()
    pltpu.sync_copy(deg_sh.at[pl.ds(s * zs, zs)],
                    degp_hbm.at[pl.ds(c * n_pad + s * zs, zs)])


def _edge_kernel(n_pad, cpw, zs, d, ei_hbm, xs_hbm, outp_hbm,
                 ei_a, ei_b, rows_a, rows_b, z_v, out_sh, ga, gb, sa, sb):
    c = lax.axis_index("c")
    s = lax.axis_index("s")

    with jax.named_scope("acc_zero"):
        def zrow(i, carry):
            def zcol(j, cc):
                z_v[i, pl.ds(j * LANES, LANES)] = jnp.zeros((LANES,),
                                                            jnp.float32)
                return cc

            lax.fori_loop(0, d // LANES, zcol, 0)
            return carry

        lax.fori_loop(0, 64, zrow, 0)

        def zcopy(t, carry):
            pltpu.sync_copy(z_v, out_sh.at[pl.ds(s * zs + t * 64, 64)])
            return carry

        lax.fori_loop(0, zs // 64, zcopy, 0)
        plsc.subcore_barrier()

    wid = s * NC + c
    gbase = wid * cpw

    def g_start(ei, rows, sem):
        for h in range(H):
            pltpu.async_copy(xs_hbm.at[ei.at[h, 0]],
                             rows.at[pl.ds(h * B, B)], sem)

    def g_wait(ei, rows, sem):
        for h in range(H):
            pltpu.make_async_copy(xs_hbm.at[ei.at[h, 0]],
                                  rows.at[pl.ds(h * B, B)], sem).wait()

    def s_start(ei, rows, sem):
        for h in range(H):
            pltpu.async_copy(rows.at[pl.ds(h * B, B)],
                             out_sh.at[ei.at[h, 1]], sem, add=True)

    def s_wait(ei, rows, sem):
        for h in range(H):
            pltpu.make_async_copy(rows.at[pl.ds(h * B, B)],
                                  out_sh.at[ei.at[h, 1]], sem).wait()

    with jax.named_scope("chunk_loop"):
        pltpu.sync_copy(ei_hbm.at[pl.ds(gbase, H)], ei_a)
        g_start(ei_a, rows_a, ga)
        pltpu.sync_copy(ei_hbm.at[pl.ds(gbase + H, H)], ei_b)
        g_start(ei_b, rows_b, gb)
        g_wait(ei_a, rows_a, ga)
        s_start(ei_a, rows_a, sa)

        def body(t, carry):
            g0 = gbase + 2 * H * t
            s_wait(ei_a, rows_a, sa)
            pltpu.sync_copy(ei_hbm.at[pl.ds(g0, H)], ei_a)
            g_start(ei_a, rows_a, ga)
            g_wait(ei_b, rows_b, gb)
            s_start(ei_b, rows_b, sb)
            s_wait(ei_b, rows_b, sb)
            pltpu.sync_copy(ei_hbm.at[pl.ds(g0 + H, H)], ei_b)
            g_start(ei_b, rows_b, gb)
            g_wait(ei_a, rows_a, ga)
            s_start(ei_a, rows_a, sa)
            return carry

        lax.fori_loop(1, cpw // (2 * H), body, 0)

        g_wait(ei_b, rows_b, gb)
        s_start(ei_b, rows_b, sb)
        s_wait(ei_a, rows_a, sa)
        s_wait(ei_b, rows_b, sb)

    with jax.named_scope("copyout"):
        plsc.subcore_barrier()
        pltpu.sync_copy(out_sh.at[pl.ds(s * zs, zs)],
                        outp_hbm.at[pl.ds(c * n_pad + s * zs, zs)])


def _dis_from_parts(dp_ref):
    deg = dp_ref[0, :] + dp_ref[1, :]
    return jnp.where(deg > 0, lax.rsqrt(jnp.maximum(deg, 1.0)), 0.0)


def _scale_kernel(dp_ref, x_ref, xs_ref):
    dis = _dis_from_parts(dp_ref)
    xs_ref[...] = x_ref[...] * dis[:, None]


def _combine_kernel(p_ref, dp_ref, o_ref):
    dis = _dis_from_parts(dp_ref)
    o_ref[...] = (p_ref[0] + p_ref[1]) * dis[:, None]


@jax.jit
def kernel(x, edge_index):
    n, d = x.shape
    e = edge_index.shape[1]

    n_pad = (n // (NS * LANES) + 1) * (NS * LANES)
    zs = n_pad // NS
    cpw = -(-e // (NW * B))
    cpw += (-cpw) % (2 * K)
    e_pad = cpw * B * NW

    pad_idx = n + (jnp.arange(e_pad - e, dtype=jnp.int32) % (n_pad - n))
    ei_p = jnp.concatenate(
        [edge_index.astype(jnp.int32),
         jnp.broadcast_to(pad_idx, (2, e_pad - e))], axis=1)
    ei = ei_p.reshape(2, NW * cpw, B).transpose(1, 0, 2)
    dstc = ei_p[1].reshape(NW * cpw, B)
    x_pad = jnp.zeros((n_pad, d), x.dtype).at[:n].set(x)

    mesh = plsc.VectorSubcoreMesh(core_axis_name="c", subcore_axis_name="s",
                                  num_cores=NC, num_subcores=NS)

    deg_parts = pl.kernel(
        functools.partial(_deg_kernel, n_pad, cpw, zs),
        out_type=jax.ShapeDtypeStruct((NC * n_pad,), jnp.float32),
        mesh=mesh,
        scratch_types=[
            pltpu.VMEM((K, B), jnp.int32),
            pltpu.VMEM((K, B), jnp.int32),
            pltpu.VMEM((B,), jnp.float32),
            pltpu.VMEM((zs,), jnp.float32),
            pltpu.VMEM_SHARED((n_pad,), jnp.float32),
            pltpu.SemaphoreType.DMA,
            pltpu.SemaphoreType.DMA,
            pltpu.SemaphoreType.DMA,
            pltpu.SemaphoreType.DMA,
        ],
    )(dstc)
    deg_parts = deg_parts.reshape(NC, n_pad)

    rb = 1024
    xs = pl.pallas_call(
        _scale_kernel,
        grid=(n_pad // rb,),
        in_specs=[
            pl.BlockSpec((NC, rb), lambda i: (0, i)),
            pl.BlockSpec((rb, d), lambda i: (i, 0)),
        ],
        out_specs=pl.BlockSpec((rb, d), lambda i: (i, 0)),
        out_shape=jax.ShapeDtypeStruct((n_pad, d), jnp.float32),
    )(deg_parts, x_pad)

    out_parts = pl.kernel(
        functools.partial(_edge_kernel, n_pad, cpw, zs, d),
        out_type=jax.ShapeDtypeStruct((NC * n_pad, d), jnp.float32),
        mesh=mesh,
        scratch_types=[
            pltpu.VMEM((H, 2, B), jnp.int32),
            pltpu.VMEM((H, 2, B), jnp.int32),
            pltpu.VMEM((H * B, d), jnp.float32),
            pltpu.VMEM((H * B, d), jnp.float32),
            pltpu.VMEM((64, d), jnp.float32),
            pltpu.VMEM_SHARED((n_pad, d), jnp.float32),
            pltpu.SemaphoreType.DMA,
            pltpu.SemaphoreType.DMA,
            pltpu.SemaphoreType.DMA,
            pltpu.SemaphoreType.DMA,
        ],
    )(ei, xs)
    out_parts = out_parts.reshape(NC, n_pad, d)

    out_pad = pl.pallas_call(
        _combine_kernel,
        grid=(n_pad // rb,),
        in_specs=[
            pl.BlockSpec((NC, rb, d), lambda i: (0, i, 0)),
            pl.BlockSpec((NC, rb), lambda i: (0, i)),
        ],
        out_specs=pl.BlockSpec((rb, d), lambda i: (i, 0)),
        out_shape=jax.ShapeDtypeStruct((n_pad, d), jnp.float32),
    )(out_parts, deg_parts)

    return out_pad[:n]
